# baseline (speedup 1.0000x reference)
import functools

import jax
import jax.numpy as jnp
from jax import lax
from jax.experimental import pallas as pl
from jax.experimental.pallas import tpu as pltpu
from jax.experimental.pallas import tpu_sc as plsc

ND = 10000
NE = 320000
DD = 128
HH = 8
DH = 16
NT = 21
NV = 1024

NC, NS = 2, 16
NWORK = NC * NS
CH = 128
NCHUNK = 80
EPW = CH * NCHUNK
NE_PAD = EPW * NWORK
ND_PAD = 10240

BE = 1024
BN = 1000
NP = 160
CPW = NP // NWORK
CHK = 512
GB = 8 * CHK
NBLK = NE_PAD // GB
NR = ND_PAD // DD

def _sc_gather2_body(x_hbm, src_hbm, dst_hbm, out_s_hbm, out_d_hbm,
                     idx0, idx1, idx2, idx3, idx4, idx5,
                     rows0, rows1, rows2, rows3, rows4, rows5,
                     sem0, sem1, sem2, sem3, sem4, sem5):
  wid = lax.axis_index("c") * NS + lax.axis_index("s")
  base = wid * EPW

  bufs = ((src_hbm, out_s_hbm, idx0, rows0, sem0),
          (src_hbm, out_s_hbm, idx1, rows1, sem1),
          (src_hbm, out_s_hbm, idx2, rows2, sem2),
          (dst_hbm, out_d_hbm, idx3, rows3, sem3),
          (dst_hbm, out_d_hbm, idx4, rows4, sem4),
          (dst_hbm, out_d_hbm, idx5, rows5, sem5))

  for ch, (ih, oh, iv, rv, sm) in zip((0, 1, 2, 0, 1, 2), bufs):
    pltpu.sync_copy(ih.at[pl.ds(base + ch * CH, CH)], iv)
    pltpu.async_copy(x_hbm.at[iv], rv, sm)

  def body(t, _):
    c0 = 3 * t
    for dch, (ih, oh, iv, rv, sm) in zip((0, 1, 2, 0, 1, 2), bufs):
      c = c0 + dch

      @pl.when(c < NCHUNK)
      def _(ih=ih, oh=oh, iv=iv, rv=rv, sm=sm, c=c):
        pltpu.make_async_copy(x_hbm.at[iv], rv, sm).wait()
        pltpu.async_copy(rv, oh.at[pl.ds(base + c * CH, CH)], sm)

        @pl.when(c + 3 < NCHUNK)
        def _():
          pltpu.make_async_copy(rv, oh.at[pl.ds(base, CH)], sm).wait()
          pltpu.sync_copy(ih.at[pl.ds(base + (c + 3) * CH, CH)], iv)
          pltpu.async_copy(x_hbm.at[iv], rv, sm)

    return 0

  lax.fori_loop(0, (NCHUNK + 2) // 3, body, 0)
  for (ih, oh, iv, rv, sm) in bufs:
    pltpu.make_async_copy(rv, oh.at[pl.ds(base, CH)], sm).wait()


def _sc_scatter_body(pay4_hbm, idx3_hbm, out_hbm,
                     idx_a, idx_b, vals_a, vals_b, acc_v, sem_a, sem_b):
  wid = lax.axis_index("c") * NS + lax.axis_index("s")
  c0 = wid * CPW

  zv = jnp.zeros((16,), jnp.float32)
  for ci in range(CPW):
    def zstep(i, _, ci=ci):
      r = i // 8
      k = i - r * 8
      acc_v[ci, r, pl.ds(k * 16, 16)] = zv
      return 0
    lax.fori_loop(0, NR * 8, zstep, 0)

  def fire(b, idx_v, vals_v, sem):
    pltpu.async_copy(idx3_hbm.at[b], idx_v, sem)
    for ci in range(CPW):
      pltpu.async_copy(pay4_hbm.at[c0 + ci, b], vals_v.at[ci], sem)

  def drain(idx_v, vals_v, sem):
    pltpu.make_async_copy(idx3_hbm.at[0], idx_v, sem).wait()
    for ci in range(CPW):
      pltpu.make_async_copy(pay4_hbm.at[c0, 0], vals_v.at[ci], sem).wait()

  def compute(b, idx_v, vals_v):
    def sstep(t, _):
      for u in range(4):
        tt = t * 4 + u
        r = tt // (CHK // 16)
        k = tt - r * (CHK // 16)
        iv = idx_v[r, pl.ds(k * 16, 16)]
        hi = iv >> 7
        lo = iv & 127
        for ci in range(CPW):
          vv = vals_v[ci, r, pl.ds(k * 16, 16)]
          plsc.addupdate_scatter(acc_v.at[ci], [hi, lo], vv)
      return 0

    lax.fori_loop(0, GB // 64, sstep, 0)

  fire(0, idx_a, vals_a, sem_a)

  def body(t, _):
    b0 = 2 * t
    drain(idx_a, vals_a, sem_a)
    pl.when(b0 + 1 < NBLK)(lambda: fire(b0 + 1, idx_b, vals_b, sem_b))
    compute(b0, idx_a, vals_a)

    @pl.when(b0 + 1 < NBLK)
    def _():
      drain(idx_b, vals_b, sem_b)
      pl.when(b0 + 2 < NBLK)(lambda: fire(b0 + 2, idx_a, vals_a, sem_a))
      compute(b0 + 1, idx_b, vals_b)

    return 0

  lax.fori_loop(0, (NBLK + 1) // 2, body, 0)

  for ci in range(CPW):
    pltpu.sync_copy(acc_v.at[ci], out_hbm.at[c0 + ci])


@functools.lru_cache(maxsize=None)
def _sc_kernels():
  mesh = plsc.VectorSubcoreMesh(
      core_axis_name="c", subcore_axis_name="s",
      num_cores=NC, num_subcores=NS)
  gather2 = pl.kernel(
      _sc_gather2_body,
      out_type=[jax.ShapeDtypeStruct((NE_PAD, DD), jnp.float32),
                jax.ShapeDtypeStruct((NE_PAD, DD), jnp.float32)],
      mesh=mesh,
      scratch_types=([pltpu.VMEM((CH,), jnp.int32)] * 6
                     + [pltpu.VMEM((CH, DD), jnp.float32)] * 6
                     + [pltpu.SemaphoreType.DMA] * 6),
  )
  scatter = pl.kernel(
      _sc_scatter_body,
      out_type=jax.ShapeDtypeStruct((NP, NR, DD), jnp.float32),
      mesh=mesh,
      compiler_params=pltpu.CompilerParams(needs_layout_passes=False),
      scratch_types=[pltpu.VMEM((8, CHK), jnp.int32),
                     pltpu.VMEM((8, CHK), jnp.int32),
                     pltpu.VMEM((CPW, 8, CHK), jnp.float32),
                     pltpu.VMEM((CPW, 8, CHK), jnp.float32),
                     pltpu.VMEM((CPW, NR, DD), jnp.float32),
                     pltpu.SemaphoreType.DMA,
                     pltpu.SemaphoreType.DMA],
  )
  return gather2, scatter


def _sc_gather2(x, src2, dst2):
  return _sc_kernels()[0](x, src2, dst2)


def _sc_scatter(pay4, idx3):
  return _sc_kernels()[1](pay4, idx3)


def _embed_body(objs_ref, emb_ref, pos_ref, out_ref):
  oh = (objs_ref[...] ==
        lax.broadcasted_iota(jnp.int32, (BN, NV), 1)).astype(jnp.float32)
  out_ref[...] = (jnp.dot(oh, emb_ref[...], preferred_element_type=jnp.float32)
                  + pos_ref[...])


def _embed(objs2, obj_emb, pos):
  return pl.pallas_call(
      _embed_body,
      grid=(ND // BN,),
      in_specs=[pl.BlockSpec((BN, 1), lambda i: (i, 0)),
                pl.BlockSpec((NV, DD), lambda i: (0, 0)),
                pl.BlockSpec((BN, DD), lambda i: (i, 0))],
      out_specs=pl.BlockSpec((BN, DD), lambda i: (i, 0)),
      out_shape=jax.ShapeDtypeStruct((ND, DD), jnp.float32),
  )(objs2, obj_emb, pos)


def _head_masks():
  d_i = lax.broadcasted_iota(jnp.int32, (DD, HH), 0)
  h_i = lax.broadcasted_iota(jnp.int32, (DD, HH), 1)
  mh = (d_i // DH == h_i).astype(jnp.float32)
  return mh, mh.T


def _edge_body(layer, xs_ref, xd_ref, et_ref, eemb_ref, wq_ref, wk_ref,
               wv_ref, we_ref, wef_ref, ef_in_ref, pay_ref, ef_out_ref):
  xs = xs_ref[...]
  xd = xd_ref[...]
  xsb = xs.astype(jnp.bfloat16)
  xdb = xd.astype(jnp.bfloat16)
  oh = (et_ref[...] ==
        lax.broadcasted_iota(jnp.int32, (BE, NT), 1)).astype(jnp.bfloat16)
  if layer == 0:
    efeat = jnp.dot(oh, eemb_ref[...], preferred_element_type=jnp.float32)
  else:
    if layer == 1:
      ef_prev = jnp.dot(oh, eemb_ref[...], preferred_element_type=jnp.float32)
    else:
      ef_prev = ef_in_ref[...]
    efeat = ef_prev + jnp.tanh(
        jnp.dot((xs + xd + ef_prev).astype(jnp.bfloat16), wef_ref[...],
                preferred_element_type=jnp.float32))
    if layer == 1:
      ef_out_ref[...] = efeat
  e = jnp.dot(efeat.astype(jnp.bfloat16), we_ref[...],
              preferred_element_type=jnp.float32)
  q = jnp.dot(xdb, wq_ref[...], preferred_element_type=jnp.float32)
  k = jnp.dot(xsb, wk_ref[...], preferred_element_type=jnp.float32)
  v = jnp.dot(xsb, wv_ref[...], preferred_element_type=jnp.float32)
  mh, mht = _head_masks()
  s = jnp.dot(q * (k + e), mh, preferred_element_type=jnp.float32) * 0.25
  ex = jnp.exp(s)
  exb = jnp.dot(ex, mht, preferred_element_type=jnp.float32)
  msg = exb * (v + e)
  pay_ref[0:DD, :] = jnp.transpose(msg)
  pay_ref[DD:DD + HH, :] = jnp.transpose(ex)
  pay_ref[DD + HH:NP, :] = jnp.zeros((NP - DD - HH, BE), jnp.float32)


def _edge(layer, xg_s, xg_d, et2, edge_emb, wq, wk, wv, we, wef, ef_in):
  wspec = pl.BlockSpec((DD, DD), lambda i: (0, 0))
  espec = pl.BlockSpec((BE, DD), lambda i: (i, 0))
  out_shapes = [jax.ShapeDtypeStruct((NP, NE_PAD), jnp.float32),
                jax.ShapeDtypeStruct((NE_PAD, DD) if layer == 1 else (8, DD),
                                     jnp.float32)]
  ef_out_spec = (espec if layer == 1
                 else pl.BlockSpec((8, DD), lambda i: (0, 0)))
  ef_in_spec = espec if layer == 2 else pl.BlockSpec((8, DD), lambda i: (0, 0))
  res = pl.pallas_call(
      functools.partial(_edge_body, layer),
      grid=(NE_PAD // BE,),
      in_specs=[espec, espec,
                pl.BlockSpec((BE, 1), lambda i: (i, 0)),
                pl.BlockSpec((NT, DD), lambda i: (0, 0)),
                wspec, wspec, wspec, wspec, wspec, ef_in_spec],
      out_specs=[pl.BlockSpec((NP, BE), lambda i: (0, i)), ef_out_spec],
      out_shape=out_shapes,
  )(xg_s, xg_d, et2, edge_emb, wq, wk, wv, we, wef, ef_in)
  return res


def _xupd_body(x_ref, nt_ref, dt_ref, wo_ref, out_ref):
  _, mht = _head_masks()
  agg_n = jnp.transpose(nt_ref[...])
  den = jnp.transpose(dt_ref[...]) + 1e-9
  denb = jnp.dot(den, mht, preferred_element_type=jnp.float32)
  agg = agg_n / denb
  out_ref[...] = x_ref[...] + jnp.dot(agg, wo_ref[...],
                                      preferred_element_type=jnp.float32)


def _xupd(x, acc, wo):
  return pl.pallas_call(
      _xupd_body,
      grid=(1,),
      in_specs=[pl.BlockSpec((ND, DD), lambda i: (0, 0)),
                pl.BlockSpec((DD, ND), lambda i: (0, 0)),
                pl.BlockSpec((HH, ND), lambda i: (0, 0)),
                pl.BlockSpec((DD, DD), lambda i: (0, 0))],
      out_specs=pl.BlockSpec((ND, DD), lambda i: (0, 0)),
      out_shape=jax.ShapeDtypeStruct((ND, DD), jnp.float32),
  )(x, acc[:DD, :ND], acc[DD:DD + HH, :ND], wo)


def _final_body(x_ref, g_ref, b_ref, hw_ref, xo_ref, lg_ref):
  x = x_ref[...]
  mu = jnp.mean(x, axis=-1, keepdims=True)
  xc = x - mu
  var = jnp.mean(xc * xc, axis=-1, keepdims=True)
  xn = xc * lax.rsqrt(var + 1e-5) * g_ref[...] + b_ref[...]
  xo_ref[...] = xn
  lg_ref[...] = jnp.dot(xn, hw_ref[...], preferred_element_type=jnp.float32)


def _final(x, ln_g, ln_b, head_wt):
  nspec = pl.BlockSpec((BN, DD), lambda i: (i, 0))
  return pl.pallas_call(
      _final_body,
      grid=(ND // BN,),
      in_specs=[nspec,
                pl.BlockSpec((1, DD), lambda i: (0, 0)),
                pl.BlockSpec((1, DD), lambda i: (0, 0)),
                pl.BlockSpec((DD, NV), lambda i: (0, 0))],
      out_specs=[nspec, pl.BlockSpec((BN, NV), lambda i: (i, 0))],
      out_shape=[jax.ShapeDtypeStruct((ND, DD), jnp.float32),
                 jax.ShapeDtypeStruct((ND, NV), jnp.float32)],
  )(x, ln_g, ln_b, head_wt)


def kernel(objs, edge_index, edge_type, obj_emb, pos_emb, edge_emb,
           Wq, Wk, Wv, Wo, We, Wef, ln_g, ln_b, head_w):
  pad = NE_PAD - NE
  src = jnp.concatenate([edge_index[0],
                         jnp.zeros((pad,), edge_index.dtype)])
  dst = jnp.concatenate([edge_index[1],
                         jnp.full((pad,), ND, edge_index.dtype)])
  src2 = src.astype(jnp.int32)
  dst2 = dst.astype(jnp.int32)
  et2 = jnp.concatenate([edge_type,
                         jnp.zeros((pad,), edge_type.dtype)])
  et2 = et2.astype(jnp.int32).reshape(NE_PAD, 1)
  objs2 = objs.reshape(ND, 1).astype(jnp.int32)
  pos = pos_emb.reshape(-1, DD)[:ND]
  idx3 = dst2.reshape(NBLK, 8, CHK)
  efeat = jnp.zeros((8, DD), jnp.float32)

  x = _embed(objs2, obj_emb, pos)
  for l in range(3):
    xg_s, xg_d = _sc_gather2(x, src2, dst2)
    wef = Wef[l - 1] if l > 0 else Wef[0]
    b16 = jnp.bfloat16
    payt, ef_out = _edge(l, xg_s, xg_d, et2, edge_emb.astype(b16),
                         Wq[l].astype(b16), Wk[l].astype(b16),
                         Wv[l].astype(b16), We[l].astype(b16),
                         wef.astype(b16), efeat)
    if l == 1:
      efeat = ef_out
    pay4 = payt.reshape(NP, NBLK, 8, CHK)
    acc = _sc_scatter(pay4, idx3).reshape(NP, ND_PAD)
    x = _xupd(x, acc, Wo[l])

  return _final(x, ln_g.reshape(1, DD), ln_b.reshape(1, DD),
                jnp.transpose(head_w))


# speedup vs baseline: 1.0056x; 1.0056x over previous
import functools

import jax
import jax.numpy as jnp
from jax import lax
from jax.experimental import pallas as pl
from jax.experimental.pallas import tpu as pltpu
from jax.experimental.pallas import tpu_sc as plsc

ND = 10000
NE = 320000
DD = 128
HH = 8
DH = 16
NT = 21
NV = 1024

NC, NS = 2, 16
NWORK = NC * NS
CH = 128
NCHUNK = 80
EPW = CH * NCHUNK
NE_PAD = EPW * NWORK
ND_PAD = 10240

BE = 1024
BN = 1000
NP = 160
CPW = NP // NWORK
CHK = 512
GB = 8 * CHK
NBLK = NE_PAD // GB
NR = ND_PAD // DD

def _sc_gather2_body(x_hbm, src_hbm, dst_hbm, out_s_hbm, out_d_hbm,
                     idx0, idx1, idx2, idx3, rows0, rows1, rows2, rows3,
                     sem0, sem1, sem2, sem3):
  wid = lax.axis_index("c") * NS + lax.axis_index("s")
  base = wid * EPW

  bufs = ((src_hbm, out_s_hbm, idx0, rows0, sem0),
          (src_hbm, out_s_hbm, idx1, rows1, sem1),
          (dst_hbm, out_d_hbm, idx2, rows2, sem2),
          (dst_hbm, out_d_hbm, idx3, rows3, sem3))

  for ch, (ih, oh, iv, rv, sm) in zip((0, 1, 0, 1), bufs):
    pltpu.sync_copy(ih.at[pl.ds(base + ch * CH, CH)], iv)
    pltpu.async_copy(x_hbm.at[iv], rv, sm)

  def body(t, _):
    c0 = 2 * t
    for dch, (ih, oh, iv, rv, sm) in zip((0, 1, 0, 1), bufs):
      c = c0 + dch
      pltpu.make_async_copy(x_hbm.at[iv], rv, sm).wait()
      pltpu.async_copy(rv, oh.at[pl.ds(base + c * CH, CH)], sm)

      @pl.when(c + 2 < NCHUNK)
      def _(ih=ih, iv=iv, rv=rv, sm=sm, c=c):
        pltpu.make_async_copy(rv, oh.at[pl.ds(base, CH)], sm).wait()
        pltpu.sync_copy(ih.at[pl.ds(base + (c + 2) * CH, CH)], iv)
        pltpu.async_copy(x_hbm.at[iv], rv, sm)

    return 0

  lax.fori_loop(0, NCHUNK // 2, body, 0)
  for (ih, oh, iv, rv, sm) in bufs:
    pltpu.make_async_copy(rv, oh.at[pl.ds(base, CH)], sm).wait()


def _sc_scatter_body(pay4_hbm, idx3_hbm, out_hbm,
                     idx_a, idx_b, vals_a, vals_b, acc_v, sem_a, sem_b):
  wid = lax.axis_index("c") * NS + lax.axis_index("s")
  c0 = wid * CPW

  zv = jnp.zeros((16,), jnp.float32)
  for ci in range(CPW):
    def zstep(i, _, ci=ci):
      r = i // 8
      k = i - r * 8
      acc_v[ci, r, pl.ds(k * 16, 16)] = zv
      return 0
    lax.fori_loop(0, NR * 8, zstep, 0)

  def fire(b, idx_v, vals_v, sem):
    pltpu.async_copy(idx3_hbm.at[b], idx_v, sem)
    for ci in range(CPW):
      pltpu.async_copy(pay4_hbm.at[c0 + ci, b], vals_v.at[ci], sem)

  def drain(idx_v, vals_v, sem):
    pltpu.make_async_copy(idx3_hbm.at[0], idx_v, sem).wait()
    for ci in range(CPW):
      pltpu.make_async_copy(pay4_hbm.at[c0, 0], vals_v.at[ci], sem).wait()

  def compute(b, idx_v, vals_v):
    def sstep(t, _):
      for u in range(4):
        tt = t * 4 + u
        r = tt // (CHK // 16)
        k = tt - r * (CHK // 16)
        iv = idx_v[r, pl.ds(k * 16, 16)]
        hi = iv >> 7
        lo = iv & 127
        for ci in range(CPW):
          vv = vals_v[ci, r, pl.ds(k * 16, 16)]
          plsc.addupdate_scatter(acc_v.at[ci], [hi, lo], vv)
      return 0

    lax.fori_loop(0, GB // 64, sstep, 0)

  fire(0, idx_a, vals_a, sem_a)

  def body(t, _):
    b0 = 2 * t
    drain(idx_a, vals_a, sem_a)
    pl.when(b0 + 1 < NBLK)(lambda: fire(b0 + 1, idx_b, vals_b, sem_b))
    compute(b0, idx_a, vals_a)

    @pl.when(b0 + 1 < NBLK)
    def _():
      drain(idx_b, vals_b, sem_b)
      pl.when(b0 + 2 < NBLK)(lambda: fire(b0 + 2, idx_a, vals_a, sem_a))
      compute(b0 + 1, idx_b, vals_b)

    return 0

  lax.fori_loop(0, (NBLK + 1) // 2, body, 0)

  for ci in range(CPW):
    pltpu.sync_copy(acc_v.at[ci], out_hbm.at[c0 + ci])


@functools.lru_cache(maxsize=None)
def _sc_kernels():
  mesh = plsc.VectorSubcoreMesh(
      core_axis_name="c", subcore_axis_name="s",
      num_cores=NC, num_subcores=NS)
  gather2 = pl.kernel(
      _sc_gather2_body,
      out_type=[jax.ShapeDtypeStruct((NE_PAD, DD), jnp.float32),
                jax.ShapeDtypeStruct((NE_PAD, DD), jnp.float32)],
      mesh=mesh,
      scratch_types=([pltpu.VMEM((CH,), jnp.int32)] * 4
                     + [pltpu.VMEM((CH, DD), jnp.float32)] * 4
                     + [pltpu.SemaphoreType.DMA] * 4),
  )
  scatter = pl.kernel(
      _sc_scatter_body,
      out_type=jax.ShapeDtypeStruct((NP, NR, DD), jnp.float32),
      mesh=mesh,
      compiler_params=pltpu.CompilerParams(needs_layout_passes=False),
      scratch_types=[pltpu.VMEM((8, CHK), jnp.int32),
                     pltpu.VMEM((8, CHK), jnp.int32),
                     pltpu.VMEM((CPW, 8, CHK), jnp.float32),
                     pltpu.VMEM((CPW, 8, CHK), jnp.float32),
                     pltpu.VMEM((CPW, NR, DD), jnp.float32),
                     pltpu.SemaphoreType.DMA,
                     pltpu.SemaphoreType.DMA],
  )
  return gather2, scatter


def _sc_gather2(x, src2, dst2):
  return _sc_kernels()[0](x, src2, dst2)


def _sc_scatter(pay4, idx3):
  return _sc_kernels()[1](pay4, idx3)


def _embed_body(objs_ref, emb_ref, pos_ref, out_ref):
  oh = (objs_ref[...] ==
        lax.broadcasted_iota(jnp.int32, (BN, NV), 1)).astype(jnp.float32)
  out_ref[...] = (jnp.dot(oh, emb_ref[...], preferred_element_type=jnp.float32)
                  + pos_ref[...])


def _embed(objs2, obj_emb, pos):
  return pl.pallas_call(
      _embed_body,
      grid=(ND // BN,),
      in_specs=[pl.BlockSpec((BN, 1), lambda i: (i, 0)),
                pl.BlockSpec((NV, DD), lambda i: (0, 0)),
                pl.BlockSpec((BN, DD), lambda i: (i, 0))],
      out_specs=pl.BlockSpec((BN, DD), lambda i: (i, 0)),
      out_shape=jax.ShapeDtypeStruct((ND, DD), jnp.float32),
  )(objs2, obj_emb, pos)


def _head_masks():
  d_i = lax.broadcasted_iota(jnp.int32, (DD, HH), 0)
  h_i = lax.broadcasted_iota(jnp.int32, (DD, HH), 1)
  mh = (d_i // DH == h_i).astype(jnp.float32)
  return mh, mh.T


def _edge_body(layer, xs_ref, xd_ref, et_ref, eemb_ref, wq_ref, wk_ref,
               wv_ref, we_ref, wef_ref, ef_in_ref, pay_ref, ef_out_ref):
  xs = xs_ref[...]
  xd = xd_ref[...]
  xsb = xs.astype(jnp.bfloat16)
  xdb = xd.astype(jnp.bfloat16)
  oh = (et_ref[...] ==
        lax.broadcasted_iota(jnp.int32, (BE, NT), 1)).astype(jnp.bfloat16)
  if layer == 0:
    efeat = jnp.dot(oh, eemb_ref[...], preferred_element_type=jnp.float32)
  else:
    if layer == 1:
      ef_prev = jnp.dot(oh, eemb_ref[...], preferred_element_type=jnp.float32)
    else:
      ef_prev = ef_in_ref[...]
    efeat = ef_prev + jnp.tanh(
        jnp.dot((xs + xd + ef_prev).astype(jnp.bfloat16), wef_ref[...],
                preferred_element_type=jnp.float32))
    if layer == 1:
      ef_out_ref[...] = efeat
  e = jnp.dot(efeat.astype(jnp.bfloat16), we_ref[...],
              preferred_element_type=jnp.float32)
  q = jnp.dot(xdb, wq_ref[...], preferred_element_type=jnp.float32)
  k = jnp.dot(xsb, wk_ref[...], preferred_element_type=jnp.float32)
  v = jnp.dot(xsb, wv_ref[...], preferred_element_type=jnp.float32)
  mh, mht = _head_masks()
  s = jnp.dot(q * (k + e), mh, preferred_element_type=jnp.float32) * 0.25
  ex = jnp.exp(s)
  exb = jnp.dot(ex, mht, preferred_element_type=jnp.float32)
  msg = exb * (v + e)
  pay_ref[0:DD, :] = jnp.transpose(msg)
  pay_ref[DD:DD + HH, :] = jnp.transpose(ex)
  pay_ref[DD + HH:NP, :] = jnp.zeros((NP - DD - HH, BE), jnp.float32)


def _edge(layer, xg_s, xg_d, et2, edge_emb, wq, wk, wv, we, wef, ef_in):
  wspec = pl.BlockSpec((DD, DD), lambda i: (0, 0))
  espec = pl.BlockSpec((BE, DD), lambda i: (i, 0))
  out_shapes = [jax.ShapeDtypeStruct((NP, NE_PAD), jnp.float32),
                jax.ShapeDtypeStruct((NE_PAD, DD) if layer == 1 else (8, DD),
                                     jnp.float32)]
  ef_out_spec = (espec if layer == 1
                 else pl.BlockSpec((8, DD), lambda i: (0, 0)))
  ef_in_spec = espec if layer == 2 else pl.BlockSpec((8, DD), lambda i: (0, 0))
  res = pl.pallas_call(
      functools.partial(_edge_body, layer),
      grid=(NE_PAD // BE,),
      in_specs=[espec, espec,
                pl.BlockSpec((BE, 1), lambda i: (i, 0)),
                pl.BlockSpec((NT, DD), lambda i: (0, 0)),
                wspec, wspec, wspec, wspec, wspec, ef_in_spec],
      out_specs=[pl.BlockSpec((NP, BE), lambda i: (0, i)), ef_out_spec],
      out_shape=out_shapes,
  )(xg_s, xg_d, et2, edge_emb, wq, wk, wv, we, wef, ef_in)
  return res


def _xupd_body(x_ref, nt_ref, dt_ref, wo_ref, out_ref):
  _, mht = _head_masks()
  agg_n = jnp.transpose(nt_ref[...])
  den = jnp.transpose(dt_ref[...]) + 1e-9
  denb = jnp.dot(den, mht, preferred_element_type=jnp.float32)
  agg = agg_n / denb
  out_ref[...] = x_ref[...] + jnp.dot(agg, wo_ref[...],
                                      preferred_element_type=jnp.float32)


def _xupd(x, acc, wo):
  return pl.pallas_call(
      _xupd_body,
      grid=(1,),
      in_specs=[pl.BlockSpec((ND, DD), lambda i: (0, 0)),
                pl.BlockSpec((DD, ND), lambda i: (0, 0)),
                pl.BlockSpec((HH, ND), lambda i: (0, 0)),
                pl.BlockSpec((DD, DD), lambda i: (0, 0))],
      out_specs=pl.BlockSpec((ND, DD), lambda i: (0, 0)),
      out_shape=jax.ShapeDtypeStruct((ND, DD), jnp.float32),
  )(x, acc[:DD, :ND], acc[DD:DD + HH, :ND], wo)


def _final_body(x_ref, g_ref, b_ref, hw_ref, xo_ref, lg_ref):
  x = x_ref[...]
  mu = jnp.mean(x, axis=-1, keepdims=True)
  xc = x - mu
  var = jnp.mean(xc * xc, axis=-1, keepdims=True)
  xn = xc * lax.rsqrt(var + 1e-5) * g_ref[...] + b_ref[...]
  xo_ref[...] = xn
  lg_ref[...] = jnp.dot(xn, hw_ref[...], preferred_element_type=jnp.float32)


def _final(x, ln_g, ln_b, head_wt):
  nspec = pl.BlockSpec((BN, DD), lambda i: (i, 0))
  return pl.pallas_call(
      _final_body,
      grid=(ND // BN,),
      in_specs=[nspec,
                pl.BlockSpec((1, DD), lambda i: (0, 0)),
                pl.BlockSpec((1, DD), lambda i: (0, 0)),
                pl.BlockSpec((DD, NV), lambda i: (0, 0))],
      out_specs=[nspec, pl.BlockSpec((BN, NV), lambda i: (i, 0))],
      out_shape=[jax.ShapeDtypeStruct((ND, DD), jnp.float32),
                 jax.ShapeDtypeStruct((ND, NV), jnp.float32)],
  )(x, ln_g, ln_b, head_wt)


def kernel(objs, edge_index, edge_type, obj_emb, pos_emb, edge_emb,
           Wq, Wk, Wv, Wo, We, Wef, ln_g, ln_b, head_w):
  pad = NE_PAD - NE
  src = jnp.concatenate([edge_index[0],
                         jnp.zeros((pad,), edge_index.dtype)])
  dst = jnp.concatenate([edge_index[1],
                         jnp.full((pad,), ND, edge_index.dtype)])
  src2 = src.astype(jnp.int32)
  dst2 = dst.astype(jnp.int32)
  et2 = jnp.concatenate([edge_type,
                         jnp.zeros((pad,), edge_type.dtype)])
  et2 = et2.astype(jnp.int32).reshape(NE_PAD, 1)
  objs2 = objs.reshape(ND, 1).astype(jnp.int32)
  pos = pos_emb.reshape(-1, DD)[:ND]
  idx3 = dst2.reshape(NBLK, 8, CHK)
  efeat = jnp.zeros((8, DD), jnp.float32)

  x = _embed(objs2, obj_emb, pos)
  for l in range(3):
    xg_s, xg_d = _sc_gather2(x, src2, dst2)
    wef = Wef[l - 1] if l > 0 else Wef[0]
    b16 = jnp.bfloat16
    payt, ef_out = _edge(l, xg_s, xg_d, et2, edge_emb.astype(b16),
                         Wq[l].astype(b16), Wk[l].astype(b16),
                         Wv[l].astype(b16), We[l].astype(b16),
                         wef.astype(b16), efeat)
    if l == 1:
      efeat = ef_out
    pay4 = payt.reshape(NP, NBLK, 8, CHK)
    acc = _sc_scatter(pay4, idx3).reshape(NP, ND_PAD)
    x = _xupd(x, acc, Wo[l])

  return _final(x, ln_g.reshape(1, DD), ln_b.reshape(1, DD),
                jnp.transpose(head_w))


# speedup vs baseline: 1.1640x; 1.1575x over previous
import functools

import jax
import jax.numpy as jnp
from jax import lax
from jax.experimental import pallas as pl
from jax.experimental.pallas import tpu as pltpu
from jax.experimental.pallas import tpu_sc as plsc

ND = 10000
NE = 320000
DD = 128
HH = 8
DH = 16
NT = 21
NV = 1024

NC, NS = 2, 16
NWORK = NC * NS
CH = 128
NCHUNK = 80
EPW = CH * NCHUNK
NE_PAD = EPW * NWORK
ND_PAD = 10240

BE = 1024
BN = 1000
NP = 160
CPW = NP // NWORK
CHK = 512
GB = 8 * CHK
NBLK = NE_PAD // GB
NR = ND_PAD // DD

def _sc_gather2_body(x_hbm, src_hbm, dst_hbm, out_s_hbm, out_d_hbm,
                     idx0, idx1, idx2, idx3, rows0, rows1, rows2, rows3,
                     sem0, sem1, sem2, sem3):
  wid = lax.axis_index("c") * NS + lax.axis_index("s")
  base = wid * EPW

  bufs = ((src_hbm, out_s_hbm, idx0, rows0, sem0),
          (src_hbm, out_s_hbm, idx1, rows1, sem1),
          (dst_hbm, out_d_hbm, idx2, rows2, sem2),
          (dst_hbm, out_d_hbm, idx3, rows3, sem3))

  for ch, (ih, oh, iv, rv, sm) in zip((0, 1, 0, 1), bufs):
    pltpu.sync_copy(ih.at[pl.ds(base + ch * CH, CH)], iv)
    pltpu.async_copy(x_hbm.at[iv], rv, sm)

  def body(t, _):
    c0 = 2 * t
    for dch, (ih, oh, iv, rv, sm) in zip((0, 1, 0, 1), bufs):
      c = c0 + dch
      pltpu.make_async_copy(x_hbm.at[iv], rv, sm).wait()
      pltpu.async_copy(rv, oh.at[pl.ds(base + c * CH, CH)], sm)

      @pl.when(c + 2 < NCHUNK)
      def _(ih=ih, iv=iv, rv=rv, sm=sm, c=c):
        pltpu.make_async_copy(rv, oh.at[pl.ds(base, CH)], sm).wait()
        pltpu.sync_copy(ih.at[pl.ds(base + (c + 2) * CH, CH)], iv)
        pltpu.async_copy(x_hbm.at[iv], rv, sm)

    return 0

  lax.fori_loop(0, NCHUNK // 2, body, 0)
  for (ih, oh, iv, rv, sm) in bufs:
    pltpu.make_async_copy(rv, oh.at[pl.ds(base, CH)], sm).wait()


def _sc_scatter_body(pay4_hbm, idx3_hbm, out_hbm,
                     idx_a, idx_b, vals_a, vals_b, acc_v, sem_a, sem_b):
  wid = lax.axis_index("c") * NS + lax.axis_index("s")
  c0 = wid * CPW

  zv = jnp.zeros((16,), jnp.float32)
  for ci in range(CPW):
    def zstep(i, _, ci=ci):
      r = i // 8
      k = i - r * 8
      acc_v[ci, r, pl.ds(k * 16, 16)] = zv
      return 0
    lax.fori_loop(0, NR * 8, zstep, 0)

  def fire(b, idx_v, vals_v, sem):
    pltpu.async_copy(idx3_hbm.at[b], idx_v, sem)
    for ci in range(CPW):
      pltpu.async_copy(pay4_hbm.at[c0 + ci, b], vals_v.at[ci], sem)

  def drain(idx_v, vals_v, sem):
    pltpu.make_async_copy(idx3_hbm.at[0], idx_v, sem).wait()
    for ci in range(CPW):
      pltpu.make_async_copy(pay4_hbm.at[c0, 0], vals_v.at[ci], sem).wait()

  def compute(b, idx_v, vals_v):
    @plsc.parallel_loop(0, GB // 16, unroll=4)
    def _(t):
      r = t // (CHK // 16)
      k = t - r * (CHK // 16)
      iv = idx_v[r, pl.ds(k * 16, 16)]
      hi = iv >> 7
      lo = iv & 127
      for ci in range(CPW):
        vv = vals_v[ci, r, pl.ds(k * 16, 16)]
        plsc.addupdate_scatter(acc_v.at[ci], [hi, lo], vv)

  fire(0, idx_a, vals_a, sem_a)

  def body(t, _):
    b0 = 2 * t
    drain(idx_a, vals_a, sem_a)
    pl.when(b0 + 1 < NBLK)(lambda: fire(b0 + 1, idx_b, vals_b, sem_b))
    compute(b0, idx_a, vals_a)

    @pl.when(b0 + 1 < NBLK)
    def _():
      drain(idx_b, vals_b, sem_b)
      pl.when(b0 + 2 < NBLK)(lambda: fire(b0 + 2, idx_a, vals_a, sem_a))
      compute(b0 + 1, idx_b, vals_b)

    return 0

  lax.fori_loop(0, (NBLK + 1) // 2, body, 0)

  for ci in range(CPW):
    pltpu.sync_copy(acc_v.at[ci], out_hbm.at[c0 + ci])


@functools.lru_cache(maxsize=None)
def _sc_kernels():
  mesh = plsc.VectorSubcoreMesh(
      core_axis_name="c", subcore_axis_name="s",
      num_cores=NC, num_subcores=NS)
  gather2 = pl.kernel(
      _sc_gather2_body,
      out_type=[jax.ShapeDtypeStruct((NE_PAD, DD), jnp.float32),
                jax.ShapeDtypeStruct((NE_PAD, DD), jnp.float32)],
      mesh=mesh,
      scratch_types=([pltpu.VMEM((CH,), jnp.int32)] * 4
                     + [pltpu.VMEM((CH, DD), jnp.float32)] * 4
                     + [pltpu.SemaphoreType.DMA] * 4),
  )
  scatter = pl.kernel(
      _sc_scatter_body,
      out_type=jax.ShapeDtypeStruct((NP, NR, DD), jnp.float32),
      mesh=mesh,
      compiler_params=pltpu.CompilerParams(needs_layout_passes=False),
      scratch_types=[pltpu.VMEM((8, CHK), jnp.int32),
                     pltpu.VMEM((8, CHK), jnp.int32),
                     pltpu.VMEM((CPW, 8, CHK), jnp.float32),
                     pltpu.VMEM((CPW, 8, CHK), jnp.float32),
                     pltpu.VMEM((CPW, NR, DD), jnp.float32),
                     pltpu.SemaphoreType.DMA,
                     pltpu.SemaphoreType.DMA],
  )
  return gather2, scatter


def _sc_gather2(x, src2, dst2):
  return _sc_kernels()[0](x, src2, dst2)


def _sc_scatter(pay4, idx3):
  return _sc_kernels()[1](pay4, idx3)


def _embed_body(objs_ref, emb_ref, pos_ref, out_ref):
  oh = (objs_ref[...] ==
        lax.broadcasted_iota(jnp.int32, (BN, NV), 1)).astype(jnp.float32)
  out_ref[...] = (jnp.dot(oh, emb_ref[...], preferred_element_type=jnp.float32)
                  + pos_ref[...])


def _embed(objs2, obj_emb, pos):
  return pl.pallas_call(
      _embed_body,
      grid=(ND // BN,),
      in_specs=[pl.BlockSpec((BN, 1), lambda i: (i, 0)),
                pl.BlockSpec((NV, DD), lambda i: (0, 0)),
                pl.BlockSpec((BN, DD), lambda i: (i, 0))],
      out_specs=pl.BlockSpec((BN, DD), lambda i: (i, 0)),
      out_shape=jax.ShapeDtypeStruct((ND, DD), jnp.float32),
  )(objs2, obj_emb, pos)


def _head_masks():
  d_i = lax.broadcasted_iota(jnp.int32, (DD, HH), 0)
  h_i = lax.broadcasted_iota(jnp.int32, (DD, HH), 1)
  mh = (d_i // DH == h_i).astype(jnp.float32)
  return mh, mh.T


def _edge_body(layer, xs_ref, xd_ref, et_ref, eemb_ref, wq_ref, wk_ref,
               wv_ref, we_ref, wef_ref, ef_in_ref, pay_ref, ef_out_ref):
  xs = xs_ref[...]
  xd = xd_ref[...]
  xsb = xs.astype(jnp.bfloat16)
  xdb = xd.astype(jnp.bfloat16)
  oh = (et_ref[...] ==
        lax.broadcasted_iota(jnp.int32, (BE, NT), 1)).astype(jnp.bfloat16)
  if layer == 0:
    efeat = jnp.dot(oh, eemb_ref[...], preferred_element_type=jnp.float32)
  else:
    if layer == 1:
      ef_prev = jnp.dot(oh, eemb_ref[...], preferred_element_type=jnp.float32)
    else:
      ef_prev = ef_in_ref[...]
    efeat = ef_prev + jnp.tanh(
        jnp.dot((xs + xd + ef_prev).astype(jnp.bfloat16), wef_ref[...],
                preferred_element_type=jnp.float32))
    if layer == 1:
      ef_out_ref[...] = efeat
  e = jnp.dot(efeat.astype(jnp.bfloat16), we_ref[...],
              preferred_element_type=jnp.float32)
  q = jnp.dot(xdb, wq_ref[...], preferred_element_type=jnp.float32)
  k = jnp.dot(xsb, wk_ref[...], preferred_element_type=jnp.float32)
  v = jnp.dot(xsb, wv_ref[...], preferred_element_type=jnp.float32)
  mh, mht = _head_masks()
  s = jnp.dot(q * (k + e), mh, preferred_element_type=jnp.float32) * 0.25
  ex = jnp.exp(s)
  exb = jnp.dot(ex, mht, preferred_element_type=jnp.float32)
  msg = exb * (v + e)
  pay_ref[0:DD, :] = jnp.transpose(msg)
  pay_ref[DD:DD + HH, :] = jnp.transpose(ex)
  pay_ref[DD + HH:NP, :] = jnp.zeros((NP - DD - HH, BE), jnp.float32)


def _edge(layer, xg_s, xg_d, et2, edge_emb, wq, wk, wv, we, wef, ef_in):
  wspec = pl.BlockSpec((DD, DD), lambda i: (0, 0))
  espec = pl.BlockSpec((BE, DD), lambda i: (i, 0))
  out_shapes = [jax.ShapeDtypeStruct((NP, NE_PAD), jnp.float32),
                jax.ShapeDtypeStruct((NE_PAD, DD) if layer == 1 else (8, DD),
                                     jnp.float32)]
  ef_out_spec = (espec if layer == 1
                 else pl.BlockSpec((8, DD), lambda i: (0, 0)))
  ef_in_spec = espec if layer == 2 else pl.BlockSpec((8, DD), lambda i: (0, 0))
  res = pl.pallas_call(
      functools.partial(_edge_body, layer),
      grid=(NE_PAD // BE,),
      in_specs=[espec, espec,
                pl.BlockSpec((BE, 1), lambda i: (i, 0)),
                pl.BlockSpec((NT, DD), lambda i: (0, 0)),
                wspec, wspec, wspec, wspec, wspec, ef_in_spec],
      out_specs=[pl.BlockSpec((NP, BE), lambda i: (0, i)), ef_out_spec],
      out_shape=out_shapes,
  )(xg_s, xg_d, et2, edge_emb, wq, wk, wv, we, wef, ef_in)
  return res


def _xupd_body(x_ref, nt_ref, dt_ref, wo_ref, out_ref):
  _, mht = _head_masks()
  agg_n = jnp.transpose(nt_ref[...])
  den = jnp.transpose(dt_ref[...]) + 1e-9
  denb = jnp.dot(den, mht, preferred_element_type=jnp.float32)
  agg = agg_n / denb
  out_ref[...] = x_ref[...] + jnp.dot(agg, wo_ref[...],
                                      preferred_element_type=jnp.float32)


def _xupd(x, acc, wo):
  return pl.pallas_call(
      _xupd_body,
      grid=(1,),
      in_specs=[pl.BlockSpec((ND, DD), lambda i: (0, 0)),
                pl.BlockSpec((DD, ND), lambda i: (0, 0)),
                pl.BlockSpec((HH, ND), lambda i: (0, 0)),
                pl.BlockSpec((DD, DD), lambda i: (0, 0))],
      out_specs=pl.BlockSpec((ND, DD), lambda i: (0, 0)),
      out_shape=jax.ShapeDtypeStruct((ND, DD), jnp.float32),
  )(x, acc[:DD, :ND], acc[DD:DD + HH, :ND], wo)


def _final_body(x_ref, g_ref, b_ref, hw_ref, xo_ref, lg_ref):
  x = x_ref[...]
  mu = jnp.mean(x, axis=-1, keepdims=True)
  xc = x - mu
  var = jnp.mean(xc * xc, axis=-1, keepdims=True)
  xn = xc * lax.rsqrt(var + 1e-5) * g_ref[...] + b_ref[...]
  xo_ref[...] = xn
  lg_ref[...] = jnp.dot(xn, hw_ref[...], preferred_element_type=jnp.float32)


def _final(x, ln_g, ln_b, head_wt):
  nspec = pl.BlockSpec((BN, DD), lambda i: (i, 0))
  return pl.pallas_call(
      _final_body,
      grid=(ND // BN,),
      in_specs=[nspec,
                pl.BlockSpec((1, DD), lambda i: (0, 0)),
                pl.BlockSpec((1, DD), lambda i: (0, 0)),
                pl.BlockSpec((DD, NV), lambda i: (0, 0))],
      out_specs=[nspec, pl.BlockSpec((BN, NV), lambda i: (i, 0))],
      out_shape=[jax.ShapeDtypeStruct((ND, DD), jnp.float32),
                 jax.ShapeDtypeStruct((ND, NV), jnp.float32)],
  )(x, ln_g, ln_b, head_wt)


def kernel(objs, edge_index, edge_type, obj_emb, pos_emb, edge_emb,
           Wq, Wk, Wv, Wo, We, Wef, ln_g, ln_b, head_w):
  pad = NE_PAD - NE
  src = jnp.concatenate([edge_index[0],
                         jnp.zeros((pad,), edge_index.dtype)])
  dst = jnp.concatenate([edge_index[1],
                         jnp.full((pad,), ND, edge_index.dtype)])
  src2 = src.astype(jnp.int32)
  dst2 = dst.astype(jnp.int32)
  et2 = jnp.concatenate([edge_type,
                         jnp.zeros((pad,), edge_type.dtype)])
  et2 = et2.astype(jnp.int32).reshape(NE_PAD, 1)
  objs2 = objs.reshape(ND, 1).astype(jnp.int32)
  pos = pos_emb.reshape(-1, DD)[:ND]
  idx3 = dst2.reshape(NBLK, 8, CHK)
  efeat = jnp.zeros((8, DD), jnp.float32)

  x = _embed(objs2, obj_emb, pos)
  for l in range(3):
    xg_s, xg_d = _sc_gather2(x, src2, dst2)
    wef = Wef[l - 1] if l > 0 else Wef[0]
    b16 = jnp.bfloat16
    payt, ef_out = _edge(l, xg_s, xg_d, et2, edge_emb.astype(b16),
                         Wq[l].astype(b16), Wk[l].astype(b16),
                         Wv[l].astype(b16), We[l].astype(b16),
                         wef.astype(b16), efeat)
    if l == 1:
      efeat = ef_out
    pay4 = payt.reshape(NP, NBLK, 8, CHK)
    acc = _sc_scatter(pay4, idx3).reshape(NP, ND_PAD)
    x = _xupd(x, acc, Wo[l])

  return _final(x, ln_g.reshape(1, DD), ln_b.reshape(1, DD),
                jnp.transpose(head_w))
